# baseline (speedup 1.0000x reference)
import jax
import jax.numpy as jnp
from jax.experimental import pallas as pl
from jax.experimental.pallas import tpu as pltpu

_ROWS_PER_BLOCK = 64


def _topk_mask_kernel(k_ref, x_ref, o_ref):
    k = k_ref[0]
    x = x_ref[...]
    bits = jax.lax.bitcast_convert_type(x, jnp.int32) & jnp.int32(0x7FFFFFFF)

    r = x.shape[0]
    c = x.shape[1]

    lo = jnp.zeros((r, 1), jnp.int32)
    hi = jnp.full((r, 1), jnp.int32(0x7F800000))
    thr = jnp.zeros((r, 1), jnp.int32)
    done = jnp.zeros((r, 1), jnp.int32)
    cl = jnp.full((r, 1), jnp.float32(c))
    ch = jnp.zeros((r, 1), jnp.float32)
    y_t = jnp.log(k.astype(jnp.float32) + 0.5)

    def probe(mid, carry):
        lo, hi, thr, done, cl, ch = carry
        cnt = c + jnp.sum((bits - mid) >> 31, axis=1, keepdims=True)
        hit = cnt == k
        thr = jnp.where((done == 0) & hit, mid, thr)
        done = jnp.maximum(done, hit.astype(jnp.int32))
        pred = cnt >= k
        cntf = cnt.astype(jnp.float32)
        lo = jnp.where(pred, mid, lo)
        cl = jnp.where(pred, cntf, cl)
        hi = jnp.where(pred, hi, mid)
        ch = jnp.where(pred, ch, cntf)
        return lo, hi, thr, done, cl, ch

    def fixed_body(mid_const, carry):
        lo, hi = carry[0], carry[1]
        cand = jnp.full((r, 1), jnp.int32(mid_const))
        bmid = lo + ((hi - lo) >> 1)
        mid = jnp.where((cand > lo) & (cand < hi), cand, bmid)
        return probe(mid, carry)

    def alt_body(i, carry):
        lo, hi, _, _, cl, ch = carry
        bmid = lo + ((hi - lo) >> 1)
        v_lo = jax.lax.bitcast_convert_type(lo, jnp.float32)
        v_hi = jax.lax.bitcast_convert_type(hi, jnp.float32)
        u_lo = v_lo * v_lo
        u_hi = v_hi * v_hi
        y_lo = jnp.log(cl + 0.5)
        y_hi = jnp.log(ch + 0.5)
        frac = (y_t - y_lo) / (y_hi - y_lo)
        v_m = jnp.sqrt(u_lo + (u_hi - u_lo) * frac)
        cand = jax.lax.bitcast_convert_type(v_m, jnp.int32)
        use = (cand > lo) & (cand < hi) & ((i & 1) == 0)
        mid = jnp.where(use, cand, bmid)
        return probe(mid, carry)

    def bisect_body(_, carry):
        lo, hi = carry[0], carry[1]
        return probe(lo + ((hi - lo) >> 1), carry)

    def maybe_more(n, body, carry):
        return jax.lax.cond(
            jnp.all(carry[3] == 1),
            lambda cr: cr,
            lambda cr: jax.lax.fori_loop(0, n, body, cr),
            carry)

    carry = (lo, hi, thr, done, cl, ch)
    carry = fixed_body(0x40280000, carry)
    carry = fixed_body(0x40380000, carry)
    carry = jax.lax.fori_loop(0, 6, alt_body, carry)
    for _ in range(3):
        carry = maybe_more(4, alt_body, carry)
    for _ in range(7):
        carry = maybe_more(4, bisect_body, carry)
    lo, hi, thr, done = carry[:4]
    thr = jnp.where(done == 1, thr, lo)

    def write_simple(_):
        o_ref[...] = jnp.where(bits >= thr, x, jnp.zeros_like(x))

    def write_tie(_):
        col = jax.lax.broadcasted_iota(jnp.int32, x.shape, 1)
        gt = bits > thr
        eq = bits == thr
        all_cols = jnp.full((r, 1), jnp.int32(c - 1))
        need = k - jnp.sum(gt.astype(jnp.int32), axis=1, keepdims=True)

        def body2(_, carry):
            lo2, hi2 = carry
            mid = lo2 + ((hi2 - lo2) >> 1)
            cnt = jnp.sum((eq & (col <= mid)).astype(jnp.int32), axis=1,
                          keepdims=True)
            pred = cnt >= need
            hi2 = jnp.where(pred, mid, hi2)
            lo2 = jnp.where(pred, lo2, mid + 1)
            return lo2, hi2

        _, hi2 = jax.lax.fori_loop(
            0, max(c - 1, 1).bit_length(),
            body2,
            (jnp.zeros((r, 1), jnp.int32), all_cols),
        )
        cutoff = jnp.where(done == 1, all_cols, hi2)
        keep = gt | (eq & (col <= cutoff))
        o_ref[...] = jnp.where(keep, x, jnp.zeros_like(x))

    jax.lax.cond(jnp.all(done == 1), write_simple, write_tie, jnp.int32(0))


def kernel(scores, k):
    n, c = scores.shape
    r = _ROWS_PER_BLOCK
    k_arr = jnp.asarray(k, jnp.int32).reshape(1)
    return pl.pallas_call(
        _topk_mask_kernel,
        grid=(n // r,),
        in_specs=[
            pl.BlockSpec(memory_space=pltpu.SMEM),
            pl.BlockSpec((r, c), lambda i: (i, 0)),
        ],
        out_specs=pl.BlockSpec((r, c), lambda i: (i, 0)),
        out_shape=jax.ShapeDtypeStruct((n, c), scores.dtype),
    )(k_arr, scores)


# speedup vs baseline: 1.2270x; 1.2270x over previous
import jax
import jax.numpy as jnp
from jax.experimental import pallas as pl
from jax.experimental.pallas import tpu as pltpu

_ROWS_PER_BLOCK = 32


def _topk_mask_kernel(k_ref, x_ref, o_ref):
    k = k_ref[0]
    x = x_ref[...]
    bits = jax.lax.bitcast_convert_type(x, jnp.int32) & jnp.int32(0x7FFFFFFF)

    r = x.shape[0]
    c = x.shape[1]

    lo = jnp.zeros((r, 1), jnp.int32)
    hi = jnp.full((r, 1), jnp.int32(0x7F800000))
    thr = jnp.zeros((r, 1), jnp.int32)
    done = jnp.zeros((r, 1), jnp.int32)
    cl = jnp.full((r, 1), jnp.float32(c))
    ch = jnp.zeros((r, 1), jnp.float32)
    y_t = jnp.log(k.astype(jnp.float32) + 0.5)

    def probe(mid, carry):
        lo, hi, thr, done, cl, ch = carry
        cnt = c + jnp.sum((bits - mid) >> 31, axis=1, keepdims=True)
        hit = cnt == k
        thr = jnp.where((done == 0) & hit, mid, thr)
        done = jnp.maximum(done, hit.astype(jnp.int32))
        pred = cnt >= k
        cntf = cnt.astype(jnp.float32)
        lo = jnp.where(pred, mid, lo)
        cl = jnp.where(pred, cntf, cl)
        hi = jnp.where(pred, hi, mid)
        ch = jnp.where(pred, ch, cntf)
        return lo, hi, thr, done, cl, ch

    def fixed_body(mid_const, carry):
        lo, hi = carry[0], carry[1]
        cand = jnp.full((r, 1), jnp.int32(mid_const))
        bmid = lo + ((hi - lo) >> 1)
        mid = jnp.where((cand > lo) & (cand < hi), cand, bmid)
        return probe(mid, carry)

    def alt_body(i, carry):
        lo, hi, _, _, cl, ch = carry
        bmid = lo + ((hi - lo) >> 1)
        v_lo = jax.lax.bitcast_convert_type(lo, jnp.float32)
        v_hi = jax.lax.bitcast_convert_type(hi, jnp.float32)
        u_lo = v_lo * v_lo
        u_hi = v_hi * v_hi
        y_lo = jnp.log(cl + 0.5)
        y_hi = jnp.log(ch + 0.5)
        frac = (y_t - y_lo) / (y_hi - y_lo)
        v_m = jnp.sqrt(u_lo + (u_hi - u_lo) * frac)
        cand = jax.lax.bitcast_convert_type(v_m, jnp.int32)
        use = (cand > lo) & (cand < hi) & ((i & 1) == 0)
        mid = jnp.where(use, cand, bmid)
        return probe(mid, carry)

    def bisect_body(_, carry):
        lo, hi = carry[0], carry[1]
        return probe(lo + ((hi - lo) >> 1), carry)

    def maybe_more(n, body, carry):
        return jax.lax.cond(
            jnp.all(carry[3] == 1),
            lambda cr: cr,
            lambda cr: jax.lax.fori_loop(0, n, body, cr),
            carry)

    carry = (lo, hi, thr, done, cl, ch)
    carry = fixed_body(0x40280000, carry)
    carry = fixed_body(0x40380000, carry)
    carry = jax.lax.fori_loop(0, 6, alt_body, carry)
    for _ in range(6):
        carry = maybe_more(2, alt_body, carry)
    for _ in range(7):
        carry = maybe_more(4, bisect_body, carry)
    lo, hi, thr, done = carry[:4]
    thr = jnp.where(done == 1, thr, lo)

    def write_simple(_):
        o_ref[...] = jnp.where(bits >= thr, x, jnp.zeros_like(x))

    def write_tie(_):
        col = jax.lax.broadcasted_iota(jnp.int32, x.shape, 1)
        gt = bits > thr
        eq = bits == thr
        all_cols = jnp.full((r, 1), jnp.int32(c - 1))
        need = k - jnp.sum(gt.astype(jnp.int32), axis=1, keepdims=True)

        def body2(_, carry):
            lo2, hi2 = carry
            mid = lo2 + ((hi2 - lo2) >> 1)
            cnt = jnp.sum((eq & (col <= mid)).astype(jnp.int32), axis=1,
                          keepdims=True)
            pred = cnt >= need
            hi2 = jnp.where(pred, mid, hi2)
            lo2 = jnp.where(pred, lo2, mid + 1)
            return lo2, hi2

        _, hi2 = jax.lax.fori_loop(
            0, max(c - 1, 1).bit_length(),
            body2,
            (jnp.zeros((r, 1), jnp.int32), all_cols),
        )
        cutoff = jnp.where(done == 1, all_cols, hi2)
        keep = gt | (eq & (col <= cutoff))
        o_ref[...] = jnp.where(keep, x, jnp.zeros_like(x))

    jax.lax.cond(jnp.all(done == 1), write_simple, write_tie, jnp.int32(0))


def kernel(scores, k):
    n, c = scores.shape
    r = _ROWS_PER_BLOCK
    k_arr = jnp.asarray(k, jnp.int32).reshape(1)
    return pl.pallas_call(
        _topk_mask_kernel,
        grid=(n // r,),
        in_specs=[
            pl.BlockSpec(memory_space=pltpu.SMEM),
            pl.BlockSpec((r, c), lambda i: (i, 0)),
        ],
        out_specs=pl.BlockSpec((r, c), lambda i: (i, 0)),
        out_shape=jax.ShapeDtypeStruct((n, c), scores.dtype),
    )(k_arr, scores)
